# baseline (speedup 1.0000x reference)
import jax
import jax.numpy as jnp
from jax import lax
from jax.experimental import pallas as pl
from jax.experimental.pallas import tpu as pltpu
from jax.experimental.pallas import tpu_sc as plsc

_B = 4
_S = 8192
_D = 1024
_N = _B * _S

_INFO = plsc.get_sparse_core_info()
_NC = _INFO.num_cores
_NS = _INFO.num_subcores
_LANES = _INFO.num_lanes
_NW = _NC * _NS
_PER_W = _N // _NW
_CHUNK = 8
_NCHUNK = _PER_W // _CHUNK
_VECS = _D // _LANES
_NG = 4
_MACRO = 32
_HALF = _MACRO // 2


def _body(x_hbm, idx_hbm, table_hbm, out_hbm, idx_v, *bufs):
    rows = bufs[0:_NG]
    xbig = bufs[_NG:_NG + 2]
    gsem = bufs[_NG + 2:2 * _NG + 2]
    xsemH = bufs[2 * _NG + 2:2 * _NG + 6]
    ssemH = bufs[2 * _NG + 6:2 * _NG + 10]

    wid = lax.axis_index("s") * _NC + lax.axis_index("c")
    base = wid * _PER_W

    pltpu.sync_copy(idx_hbm.at[pl.ds(base, _PER_W)], idx_v)

    def issue_gather(k, b):
        pltpu.async_copy(
            table_hbm.at[idx_v.at[pl.ds(k * _CHUNK, _CHUNK)]], rows[b],
            gsem[b])

    def issue_xload_half(row0, p, h):
        pltpu.async_copy(
            x_hbm.at[pl.ds(row0 + h * _HALF, _HALF), :],
            xbig[p].at[pl.ds(h * _HALF, _HALF), :], xsemH[2 * h + p])

    def issue_store_half(row0, p, h):
        pltpu.async_copy(
            xbig[p].at[pl.ds(h * _HALF, _HALF), :],
            out_hbm.at[pl.ds(row0 + h * _HALF, _HALF), :], ssemH[2 * h + p])

    def drain_store_half(p, h):
        pltpu.make_async_copy(
            xbig[p].at[pl.ds(h * _HALF, _HALF), :],
            out_hbm.at[pl.ds(base, _HALF), :], ssemH[2 * h + p]).wait()

    def wait_xload_half(p, h):
        pltpu.make_async_copy(
            x_hbm.at[pl.ds(base, _HALF), :],
            xbig[p].at[pl.ds(h * _HALF, _HALF), :], xsemH[2 * h + p]).wait()

    issue_xload_half(base, 0, 0)
    issue_xload_half(base, 0, 1)
    for kk in range(_NG - 1):
        issue_gather(kk, kk)

    @pl.loop(0, _NCHUNK, step=8)
    def _ring(g):
        for mm in range(8):
            b = mm % 4
            p = mm // 4
            q = 1 - p
            k = g + mm
            mrow = base + g * _CHUNK + p * _MACRO

            if b == 0:
                @pl.when(k >= 4)
                def _d0():
                    drain_store_half(q, 0)

                @pl.when(k + 4 < _NCHUNK)
                def _x0():
                    issue_xload_half(mrow + _MACRO, q, 0)
            if b == 2:
                @pl.when(k >= 4)
                def _d1():
                    drain_store_half(q, 1)

                @pl.when(k + 4 < _NCHUNK)
                def _x1():
                    issue_xload_half(mrow + _MACRO, q, 1)

            @pl.when(k + _NG - 1 < _NCHUNK)
            def _g():
                issue_gather(k + _NG - 1, (b + _NG - 1) % _NG)

            pltpu.make_async_copy(
                table_hbm.at[idx_v.at[pl.ds(0, _CHUNK)]], rows[b],
                gsem[b]).wait()
            if b == 0:
                wait_xload_half(p, 0)
            if b == 2:
                wait_xload_half(p, 1)

            @pl.loop(0, _CHUNK)
            def _row(j):
                for l in range(_VECS):
                    sl = pl.ds(l * _LANES, _LANES)
                    xbig[p][b * _CHUNK + j, sl] = (
                        xbig[p][b * _CHUNK + j, sl] + rows[b][j, sl])

            if b == 1:
                issue_store_half(mrow, p, 0)
            if b == 3:
                issue_store_half(mrow, p, 1)

    drain_store_half(1, 0)
    drain_store_half(1, 1)


@jax.jit
def _run(x2d, idx, table):
    mesh = plsc.VectorSubcoreMesh(core_axis_name="c", subcore_axis_name="s")
    return pl.kernel(
        _body,
        out_type=jax.ShapeDtypeStruct((_N, _D), jnp.float32),
        mesh=mesh,
        scratch_types=(
            [pltpu.VMEM((_PER_W,), jnp.int32)]
            + [pltpu.VMEM((_CHUNK, _D), jnp.float32)] * _NG
            + [pltpu.VMEM((_MACRO, _D), jnp.float32)] * 2
            + [pltpu.SemaphoreType.DMA] * (_NG + 8)
        ),
    )(x2d, idx, table)


def kernel(x, positional_ids, table):
    x2d = x.reshape(_N, _D)
    idx = positional_ids.reshape(_N).astype(jnp.int32)
    out = _run(x2d, idx, table)
    return out.reshape(_B, _S, _D)


# speedup vs baseline: 1.0240x; 1.0240x over previous
import jax
import jax.numpy as jnp
from jax import lax
from jax.experimental import pallas as pl
from jax.experimental.pallas import tpu as pltpu
from jax.experimental.pallas import tpu_sc as plsc

_B = 4
_S = 8192
_D = 1024
_N = _B * _S

_INFO = plsc.get_sparse_core_info()
_NC = _INFO.num_cores
_NS = _INFO.num_subcores
_LANES = _INFO.num_lanes
_NW = _NC * _NS
_PER_W = _N // _NW
_CHUNK = 8
_NCHUNK = _PER_W // _CHUNK
_VECS = _D // _LANES
_NBUF = 4


def _body(x_hbm, idx_hbm, table_hbm, out_hbm, idx_v, *bufs):
    rows = bufs[0:_NBUF]
    xb = bufs[_NBUF:2 * _NBUF]
    gsem = bufs[2 * _NBUF:3 * _NBUF]
    xsem = bufs[3 * _NBUF:4 * _NBUF]
    ssem = bufs[4 * _NBUF:5 * _NBUF]

    wid = lax.axis_index("s") * _NC + lax.axis_index("c")
    base = wid * _PER_W

    pltpu.sync_copy(idx_hbm.at[pl.ds(base, _PER_W)], idx_v)

    def issue_gather(k, b):
        pltpu.async_copy(
            table_hbm.at[idx_v.at[pl.ds(k * _CHUNK, _CHUNK)]], rows[b],
            gsem[b])

    def issue_xload(k, b):
        pltpu.async_copy(
            x_hbm.at[pl.ds(base + k * _CHUNK, _CHUNK), :], xb[b], xsem[b])

    def drain_store(k, b):
        pltpu.make_async_copy(
            xb[b], out_hbm.at[pl.ds(base + k * _CHUNK, _CHUNK), :],
            ssem[b]).wait()

    for kk in range(_NBUF - 1):
        issue_gather(kk, kk)
    for kk in range(_NBUF - 2):
        issue_xload(kk, kk)

    @pl.loop(0, _NCHUNK, step=_NBUF)
    def _ring(g):
        for b in range(_NBUF):
            k = g + b

            @pl.when(k + _NBUF - 1 < _NCHUNK)
            def _prefetch_gather():
                issue_gather(k + _NBUF - 1, (b + _NBUF - 1) % _NBUF)

            @pl.when(k - 2 >= 0)
            def _drain():
                drain_store(k - 2, (b + 2) % _NBUF)

            @pl.when(k + 2 < _NCHUNK)
            def _prefetch_xload():
                issue_xload(k + 2, (b + 2) % _NBUF)

            pltpu.make_async_copy(
                table_hbm.at[idx_v.at[pl.ds(0, _CHUNK)]], rows[b],
                gsem[b]).wait()
            pltpu.make_async_copy(
                x_hbm.at[pl.ds(base, _CHUNK), :], xb[b], xsem[b]).wait()

            @pl.loop(0, _CHUNK)
            def _row(j):
                for l in range(_VECS):
                    sl = pl.ds(l * _LANES, _LANES)
                    xb[b][j, sl] = xb[b][j, sl] + rows[b][j, sl]

            pltpu.async_copy(
                xb[b], out_hbm.at[pl.ds(base + k * _CHUNK, _CHUNK), :],
                ssem[b])

    for kk in range(_NCHUNK - 2, _NCHUNK):
        drain_store(kk, kk % _NBUF)


@jax.jit
def _run(x2d, idx, table):
    mesh = plsc.VectorSubcoreMesh(core_axis_name="c", subcore_axis_name="s")
    return pl.kernel(
        _body,
        out_type=jax.ShapeDtypeStruct((_N, _D), jnp.float32),
        mesh=mesh,
        scratch_types=(
            [pltpu.VMEM((_PER_W,), jnp.int32)]
            + [pltpu.VMEM((_CHUNK, _D), jnp.float32)] * (2 * _NBUF)
            + [pltpu.SemaphoreType.DMA] * (3 * _NBUF)
        ),
    )(x2d, idx, table)


def kernel(x, positional_ids, table):
    x2d = x.reshape(_N, _D)
    idx = positional_ids.reshape(_N).astype(jnp.int32)
    out = _run(x2d, idx, table)
    return out.reshape(_B, _S, _D)
